# Initial kernel scaffold; baseline (speedup 1.0000x reference)
import jax
import jax.numpy as jnp
from jax.experimental import pallas as pl


def kernel(edge_attr, W0, W1):
    raise NotImplementedError("write your pallas kernel here")


# speedup vs baseline: 1.7208x; 1.7208x over previous
import functools

import jax
import jax.numpy as jnp
from jax import lax
from jax.experimental import pallas as pl
from jax.experimental.pallas import tpu as pltpu
from jax.experimental.pallas import tpu_sc as plsc

EMB = 16
N_EDGES = 1600000
CHUNK = 512
NUM_CHUNKS = N_EDGES // CHUNK
NW = 32
L = 16


def _sc_lookup(idx0, idx1, Wc):
    mesh = plsc.VectorSubcoreMesh(core_axis_name="c", subcore_axis_name="s")

    @functools.partial(
        pl.kernel,
        mesh=mesh,
        compiler_params=pltpu.CompilerParams(use_tc_tiling_on_sc=False),
        out_type=jax.ShapeDtypeStruct((N_EDGES, 2 * EMB), jnp.float32),
        scratch_types=[
            pltpu.VMEM((CHUNK,), jnp.int32),
            pltpu.VMEM((CHUNK,), jnp.int32),
            pltpu.VMEM((CHUNK,), jnp.int32),
            pltpu.VMEM((CHUNK, 2 * EMB), jnp.float32),
            pltpu.SemaphoreType.DMA,
        ],
    )
    def k(idx0_hbm, idx1_hbm, wc_hbm, out_hbm, i0_v, i1_v, ci_v, out_v, sem):
        wid = lax.axis_index("s") * 2 + lax.axis_index("c")
        steps = (NUM_CHUNKS + NW - 1) // NW

        def body(t, carry):
            g = wid + t * NW

            @pl.when(g < NUM_CHUNKS)
            def _():
                base = g * CHUNK
                pltpu.sync_copy(idx0_hbm.at[pl.ds(base, CHUNK)], i0_v)
                pltpu.sync_copy(idx1_hbm.at[pl.ds(base, CHUNK)], i1_v)
                for o in range(0, CHUNK, L):
                    ci_v[pl.ds(o, L)] = i0_v[pl.ds(o, L)] * 4 + i1_v[pl.ds(o, L)]
                cps = []
                for j in range(0, CHUNK, 128):
                    cps.append(pltpu.async_copy(
                        wc_hbm.at[ci_v.at[pl.ds(j, 128)]],
                        out_v.at[pl.ds(j, 128), :], sem))
                for cp in cps:
                    cp.wait()
                pltpu.sync_copy(out_v, out_hbm.at[pl.ds(base, CHUNK), :])

            return carry

        lax.fori_loop(0, steps, body, 0)

    return k(idx0, idx1, Wc)


def kernel(edge_attr, W0, W1):
    idx0 = edge_attr[:, 0]
    idx1 = edge_attr[:, 1]
    Wc = jnp.concatenate(
        [jnp.repeat(W0, 4, axis=0), jnp.tile(W1, (4, 1))], axis=1)
    return _sc_lookup(idx0, idx1, Wc)
